# baseline (speedup 1.0000x reference)
import functools

import jax
import jax.numpy as jnp
from jax import lax
from jax.experimental import pallas as pl
from jax.experimental.pallas import tpu as pltpu
from jax.experimental.pallas import tpu_sc as plsc

_K = 8
_B = 4096
_L = 16
_NC = 2
_NS = 16
_NW = _NC * _NS
_BPW = _B // _NW
_H1 = _BPW * _K
_H2 = _H1 * _K

_MESH = plsc.VectorSubcoreMesh(core_axis_name="c", subcore_axis_name="s")
_PARAMS = pltpu.CompilerParams(use_tc_tiling_on_sc=False)


@functools.partial(
    pl.kernel,
    mesh=_MESH,
    compiler_params=_PARAMS,
    out_type=[
        jax.ShapeDtypeStruct((_B * _K,), jnp.int32),
        jax.ShapeDtypeStruct((_B * _K * _K,), jnp.int32),
        jax.ShapeDtypeStruct((_B * _K,), jnp.int32),
        jax.ShapeDtypeStruct((_B * _K * _K,), jnp.int32),
    ],
    scratch_types=[
        pltpu.VMEM((_BPW,), jnp.int32),
        pltpu.VMEM((_H1,), jnp.int32),
        pltpu.VMEM((_H1,), jnp.int32),
        pltpu.VMEM((_H1,), jnp.int32),
        pltpu.VMEM((_H2,), jnp.int32),
        pltpu.VMEM((_H2,), jnp.int32),
        pltpu.VMEM((_H2,), jnp.int32),
        pltpu.SemaphoreType.DMA,
        pltpu.SemaphoreType.DMA,
        pltpu.SemaphoreType.DMA,
    ],
)
def _encode(ent_hbm, adj_ef_hbm, adj_rf_hbm,
            ent1_hbm, ent2_hbm, rel1_hbm, rel2_hbm,
            idx_v, eo1_v, e1_v, r1_v, eo2_v, e2_v, r2_v,
            sem_a, sem_b, sem_w):
    wid = lax.axis_index("s") * _NC + lax.axis_index("c")
    base1 = pl.multiple_of(wid * _H1, 8)
    base2 = pl.multiple_of(wid * _H2, 8)

    pltpu.sync_copy(ent_hbm.at[pl.ds(pl.multiple_of(wid * _BPW, 8), _BPW)],
                    idx_v)

    lanes = lax.iota(jnp.int32, _L)
    sub = lanes >> 3
    offs = lanes & 7
    for tt in range(_BPW // _L):
        e16 = idx_v[pl.ds(tt * _L, _L)]
        for v in range(_L // 2):
            vec = jnp.where(sub == 0, e16[2 * v], e16[2 * v + 1])
            eo1_v[pl.ds(tt * (_L * _K) + v * _L, _L)] = vec * _K + offs

    c_e1 = pltpu.async_copy(adj_ef_hbm.at[eo1_v], e1_v, sem_a)
    c_r1 = pltpu.async_copy(adj_rf_hbm.at[eo1_v], r1_v, sem_b)
    c_e1.wait()

    def expand2(t, carry):
        e16 = e1_v[pl.ds(t * _L, _L)]
        for v in range(_L // 2):
            vec = jnp.where(sub == 0, e16[2 * v], e16[2 * v + 1])
            dst = pl.multiple_of(t * (_L * _K) + v * _L, 16)
            eo2_v[pl.ds(dst, _L)] = vec * _K + offs
        return carry
    lax.fori_loop(0, _H1 // _L, expand2, 0)

    c_e2 = pltpu.async_copy(adj_ef_hbm.at[eo2_v], e2_v, sem_a)
    c_r2 = pltpu.async_copy(adj_rf_hbm.at[eo2_v], r2_v, sem_b)

    w_e1 = pltpu.async_copy(e1_v, ent1_hbm.at[pl.ds(base1, _H1)], sem_w)
    c_r1.wait()
    w_r1 = pltpu.async_copy(r1_v, rel1_hbm.at[pl.ds(base1, _H1)], sem_w)

    c_e2.wait()
    w_e2 = pltpu.async_copy(e2_v, ent2_hbm.at[pl.ds(base2, _H2)], sem_w)
    c_r2.wait()
    w_r2 = pltpu.async_copy(r2_v, rel2_hbm.at[pl.ds(base2, _H2)], sem_w)

    w_e1.wait()
    w_r1.wait()
    w_e2.wait()
    w_r2.wait()


def kernel(entity, adj_entity, adj_relation):
    ent1, ent2, rel1, rel2 = _encode(
        entity.reshape(-1), adj_entity.reshape(-1), adj_relation.reshape(-1))
    return (
        entity,
        ent1.reshape(_B, _K),
        ent2.reshape(_B, _K * _K),
        rel1.reshape(_B, _K),
        rel2.reshape(_B, _K * _K),
    )


# speedup vs baseline: 2.8711x; 2.8711x over previous
import functools

import jax
import jax.numpy as jnp
from jax import lax
from jax.experimental import pallas as pl
from jax.experimental.pallas import tpu as pltpu
from jax.experimental.pallas import tpu_sc as plsc

_N = 100000
_K = 8
_B = 4096
_L = 16
_NC = 2
_NS = 16
_NW = _NC * _NS
_BPW = _B // _NW
_H1 = _BPW * _K
_H2 = _H1 * _K

_MESH = plsc.VectorSubcoreMesh(core_axis_name="c", subcore_axis_name="s")
_PARAMS = pltpu.CompilerParams(use_tc_tiling_on_sc=False)


@functools.partial(
    pl.kernel,
    mesh=_MESH,
    compiler_params=_PARAMS,
    out_type=[
        jax.ShapeDtypeStruct((_B * _K,), jnp.int32),
        jax.ShapeDtypeStruct((_B * _K * _K,), jnp.int32),
        jax.ShapeDtypeStruct((_B * _K,), jnp.int32),
        jax.ShapeDtypeStruct((_B * _K * _K,), jnp.int32),
    ],
    scratch_types=[
        pltpu.VMEM((_BPW,), jnp.int32),
        pltpu.VMEM((_H1,), jnp.int32),
        pltpu.VMEM((_H1,), jnp.int32),
        pltpu.VMEM((_H1,), jnp.int32),
        pltpu.VMEM((_H2,), jnp.int32),
        pltpu.VMEM((_H2,), jnp.int32),
        pltpu.VMEM((_H2,), jnp.int32),
        pltpu.SemaphoreType.DMA,
        pltpu.SemaphoreType.DMA,
        pltpu.SemaphoreType.DMA,
    ],
)
def _encode(ent_hbm, adj_ef_hbm, adj_rf_hbm,
            ent1_hbm, ent2_hbm, rel1_hbm, rel2_hbm,
            idx_v, eo1_v, e1_v, r1_v, eo2_v, e2_v, r2_v,
            sem_a, sem_b, sem_w):
    wid = lax.axis_index("s") * _NC + lax.axis_index("c")
    base1 = pl.multiple_of(wid * _H1, 8)
    base2 = pl.multiple_of(wid * _H2, 8)

    pltpu.sync_copy(ent_hbm.at[pl.ds(pl.multiple_of(wid * _BPW, 8), _BPW)],
                    idx_v)

    lanes = lax.iota(jnp.int32, _L)
    sub = lanes >> 3
    joff = (lanes & 7) * _N
    for tt in range(_BPW // _L):
        e16 = idx_v[pl.ds(tt * _L, _L)]
        for v in range(_L // 2):
            vec = jnp.where(sub == 0, e16[2 * v], e16[2 * v + 1])
            eo1_v[pl.ds(tt * (_L * _K) + v * _L, _L)] = vec + joff

    c_e1 = pltpu.async_copy(adj_ef_hbm.at[eo1_v], e1_v, sem_a)
    c_r1 = pltpu.async_copy(adj_rf_hbm.at[eo1_v], r1_v, sem_b)
    c_e1.wait()

    def expand2(t, carry):
        e16 = e1_v[pl.ds(t * _L, _L)]
        for v in range(_L // 2):
            vec = jnp.where(sub == 0, e16[2 * v], e16[2 * v + 1])
            dst = pl.multiple_of(t * (_L * _K) + v * _L, 16)
            eo2_v[pl.ds(dst, _L)] = vec + joff
        return carry
    lax.fori_loop(0, _H1 // _L, expand2, 0)

    c_e2 = pltpu.async_copy(adj_ef_hbm.at[eo2_v], e2_v, sem_a)
    c_r2 = pltpu.async_copy(adj_rf_hbm.at[eo2_v], r2_v, sem_b)

    w_e1 = pltpu.async_copy(e1_v, ent1_hbm.at[pl.ds(base1, _H1)], sem_w)
    c_r1.wait()
    w_r1 = pltpu.async_copy(r1_v, rel1_hbm.at[pl.ds(base1, _H1)], sem_w)

    c_e2.wait()
    w_e2 = pltpu.async_copy(e2_v, ent2_hbm.at[pl.ds(base2, _H2)], sem_w)
    c_r2.wait()
    w_r2 = pltpu.async_copy(r2_v, rel2_hbm.at[pl.ds(base2, _H2)], sem_w)

    w_e1.wait()
    w_r1.wait()
    w_e2.wait()
    w_r2.wait()


def kernel(entity, adj_entity, adj_relation):
    ent1, ent2, rel1, rel2 = _encode(
        entity.reshape(-1), adj_entity.T.reshape(-1), adj_relation.T.reshape(-1))
    return (
        entity,
        ent1.reshape(_B, _K),
        ent2.reshape(_B, _K * _K),
        rel1.reshape(_B, _K),
        rel2.reshape(_B, _K * _K),
    )


# speedup vs baseline: 2.8882x; 1.0060x over previous
import functools

import jax
import jax.numpy as jnp
from jax import lax
from jax.experimental import pallas as pl
from jax.experimental.pallas import tpu as pltpu
from jax.experimental.pallas import tpu_sc as plsc

_N = 100000
_K = 8
_B = 4096
_L = 16
_NC = 2
_NS = 16
_NW = _NC * _NS
_BPW = _B // _NW
_H1 = _BPW * _K
_H2 = _H1 * _K

_MESH = plsc.VectorSubcoreMesh(core_axis_name="c", subcore_axis_name="s")
_PARAMS = pltpu.CompilerParams(use_tc_tiling_on_sc=False)


@functools.partial(
    pl.kernel,
    mesh=_MESH,
    compiler_params=_PARAMS,
    out_type=[
        jax.ShapeDtypeStruct((_B * _K,), jnp.int32),
        jax.ShapeDtypeStruct((_B * _K * _K,), jnp.int32),
        jax.ShapeDtypeStruct((_B * _K,), jnp.int32),
        jax.ShapeDtypeStruct((_B * _K * _K,), jnp.int32),
    ],
    scratch_types=[
        pltpu.VMEM((_BPW,), jnp.int32),
        pltpu.VMEM((_H1,), jnp.int32),
        pltpu.VMEM((_H1,), jnp.int32),
        pltpu.VMEM((_H1,), jnp.int32),
        pltpu.VMEM((_H2,), jnp.int32),
        pltpu.VMEM((_H2,), jnp.int32),
        pltpu.VMEM((_H2,), jnp.int32),
        pltpu.SemaphoreType.DMA,
        pltpu.SemaphoreType.DMA,
        pltpu.SemaphoreType.DMA,
    ],
)
def _encode(ent_hbm, adj_ef_hbm, adj_rf_hbm,
            ent1_hbm, ent2_hbm, rel1_hbm, rel2_hbm,
            idx_v, eo1_v, e1_v, r1_v, eo2_v, e2_v, r2_v,
            sem_a, sem_b, sem_w):
    wid = lax.axis_index("s") * _NC + lax.axis_index("c")
    base1 = pl.multiple_of(wid * _H1, 8)
    base2 = pl.multiple_of(wid * _H2, 8)

    pltpu.sync_copy(ent_hbm.at[pl.ds(pl.multiple_of(wid * _BPW, 8), _BPW)],
                    idx_v)

    lanes = lax.iota(jnp.int32, _L)
    sub = lanes >> 3
    joff = (lanes & 7) * _N
    for tt in range(_BPW // _L):
        e16 = idx_v[pl.ds(tt * _L, _L)]
        for v in range(_L // 2):
            vec = jnp.where(sub == 0, e16[2 * v], e16[2 * v + 1])
            eo1_v[pl.ds(tt * (_L * _K) + v * _L, _L)] = vec + joff

    _HH = _H1 // 2
    c_e1a = pltpu.async_copy(
        adj_ef_hbm.at[eo1_v.at[pl.ds(0, _HH)]], e1_v.at[pl.ds(0, _HH)], sem_a)
    c_e1b = pltpu.async_copy(
        adj_ef_hbm.at[eo1_v.at[pl.ds(_HH, _HH)]], e1_v.at[pl.ds(_HH, _HH)],
        sem_a)
    c_r1 = pltpu.async_copy(adj_rf_hbm.at[eo1_v], r1_v, sem_b)

    def expand2(t, carry):
        e16 = e1_v[pl.ds(t * _L, _L)]
        for v in range(_L // 2):
            vec = jnp.where(sub == 0, e16[2 * v], e16[2 * v + 1])
            dst = pl.multiple_of(t * (_L * _K) + v * _L, 16)
            eo2_v[pl.ds(dst, _L)] = vec + joff
        return carry

    _HQ = _H2 // 2
    c_e1a.wait()
    lax.fori_loop(0, _HH // _L, expand2, 0)
    c_e2a = pltpu.async_copy(
        adj_ef_hbm.at[eo2_v.at[pl.ds(0, _HQ)]], e2_v.at[pl.ds(0, _HQ)], sem_a)
    c_r2a = pltpu.async_copy(
        adj_rf_hbm.at[eo2_v.at[pl.ds(0, _HQ)]], r2_v.at[pl.ds(0, _HQ)], sem_b)

    c_e1b.wait()
    lax.fori_loop(_HH // _L, _H1 // _L, expand2, 0)
    c_e2b = pltpu.async_copy(
        adj_ef_hbm.at[eo2_v.at[pl.ds(_HQ, _HQ)]], e2_v.at[pl.ds(_HQ, _HQ)],
        sem_a)
    c_r2b = pltpu.async_copy(
        adj_rf_hbm.at[eo2_v.at[pl.ds(_HQ, _HQ)]], r2_v.at[pl.ds(_HQ, _HQ)],
        sem_b)

    w_e1 = pltpu.async_copy(e1_v, ent1_hbm.at[pl.ds(base1, _H1)], sem_w)
    c_r1.wait()
    w_r1 = pltpu.async_copy(r1_v, rel1_hbm.at[pl.ds(base1, _H1)], sem_w)

    c_e2a.wait()
    c_e2b.wait()
    w_e2 = pltpu.async_copy(e2_v, ent2_hbm.at[pl.ds(base2, _H2)], sem_w)
    c_r2a.wait()
    c_r2b.wait()
    w_r2 = pltpu.async_copy(r2_v, rel2_hbm.at[pl.ds(base2, _H2)], sem_w)

    w_e1.wait()
    w_r1.wait()
    w_e2.wait()
    w_r2.wait()


def kernel(entity, adj_entity, adj_relation):
    ent1, ent2, rel1, rel2 = _encode(
        entity.reshape(-1), adj_entity.T.reshape(-1), adj_relation.T.reshape(-1))
    return (
        entity,
        ent1.reshape(_B, _K),
        ent2.reshape(_B, _K * _K),
        rel1.reshape(_B, _K),
        rel2.reshape(_B, _K * _K),
    )


# speedup vs baseline: 3.0684x; 1.0624x over previous
import functools

import jax
import jax.numpy as jnp
from jax import lax
from jax.experimental import pallas as pl
from jax.experimental.pallas import tpu as pltpu
from jax.experimental.pallas import tpu_sc as plsc

_N = 100000
_K = 8
_B = 4096
_L = 16
_NC = 2
_NS = 16
_NW = _NC * _NS
_BPW = _B // _NW
_C2 = _K * _K

_MESH = plsc.VectorSubcoreMesh(core_axis_name="c", subcore_axis_name="s")
_PARAMS = pltpu.CompilerParams(use_tc_tiling_on_sc=False)


@functools.partial(
    pl.kernel,
    mesh=_MESH,
    compiler_params=_PARAMS,
    out_type=[
        jax.ShapeDtypeStruct((_K, _B), jnp.int32),
        jax.ShapeDtypeStruct((_C2, _B), jnp.int32),
        jax.ShapeDtypeStruct((_K, _B), jnp.int32),
        jax.ShapeDtypeStruct((_C2, _B), jnp.int32),
    ],
    scratch_types=[
        pltpu.VMEM((_BPW,), jnp.int32),
        pltpu.VMEM((_K, _BPW), jnp.int32),
        pltpu.VMEM((_K, _BPW), jnp.int32),
        pltpu.VMEM((_K, _BPW), jnp.int32),
        pltpu.VMEM((_C2, _BPW), jnp.int32),
        pltpu.VMEM((_C2, _BPW), jnp.int32),
        pltpu.VMEM((_C2, _BPW), jnp.int32),
        pltpu.SemaphoreType.DMA,
        pltpu.SemaphoreType.DMA,
        pltpu.SemaphoreType.DMA,
    ],
)
def _encode(ent_hbm, adj_ef_hbm, adj_rf_hbm,
            ent1_hbm, ent2_hbm, rel1_hbm, rel2_hbm,
            idx_v, eo1_v, e1_v, r1_v, eo2_v, e2_v, r2_v,
            sem_a, sem_b, sem_w):
    wid = lax.axis_index("s") * _NC + lax.axis_index("c")
    base = pl.multiple_of(wid * _BPW, 8)

    pltpu.sync_copy(ent_hbm.at[pl.ds(base, _BPW)], idx_v)

    for t in range(_BPW // _L):
        e16 = idx_v[pl.ds(t * _L, _L)]
        for j in range(_K):
            eo1_v[j, pl.ds(t * _L, _L)] = e16 + (_N * j)

    h1 = []
    for j in range(_K):
        h1.append(pltpu.async_copy(
            adj_ef_hbm.at[eo1_v.at[j]], e1_v.at[j], sem_a))
        h1.append(pltpu.async_copy(
            adj_rf_hbm.at[eo1_v.at[j]], r1_v.at[j], sem_b))
    for cp in h1:
        cp.wait()

    w_e1 = pltpu.async_copy(e1_v, ent1_hbm.at[:, pl.ds(base, _BPW)], sem_w)
    w_r1 = pltpu.async_copy(r1_v, rel1_hbm.at[:, pl.ds(base, _BPW)], sem_w)

    def expand2(jt, carry):
        j = jt // (_BPW // _L)
        t = jt % (_BPW // _L)
        e16 = e1_v[j, pl.ds(pl.multiple_of(t * _L, _L), _L)]
        for k in range(_K):
            eo2_v[j * _K + k, pl.ds(pl.multiple_of(t * _L, _L), _L)] = (
                e16 + (_N * k))
        return carry
    lax.fori_loop(0, _K * (_BPW // _L), expand2, 0)

    def fire(c, carry):
        ic = eo2_v.at[c]
        pltpu.async_copy(adj_ef_hbm.at[ic], e2_v.at[c], sem_a)
        pltpu.async_copy(adj_rf_hbm.at[ic], r2_v.at[c], sem_b)
        return carry
    lax.fori_loop(0, _C2, fire, 0)

    def drain(c, carry):
        pltpu.make_async_copy(
            adj_ef_hbm.at[pl.ds(0, _BPW)], e2_v.at[c], sem_a).wait()
        pltpu.make_async_copy(
            adj_rf_hbm.at[pl.ds(0, _BPW)], r2_v.at[c], sem_b).wait()
        return carry
    lax.fori_loop(0, _C2, drain, 0)

    w_e2 = pltpu.async_copy(e2_v, ent2_hbm.at[:, pl.ds(base, _BPW)], sem_w)
    w_r2 = pltpu.async_copy(r2_v, rel2_hbm.at[:, pl.ds(base, _BPW)], sem_w)

    w_e1.wait()
    w_r1.wait()
    w_e2.wait()
    w_r2.wait()


def kernel(entity, adj_entity, adj_relation):
    ent1, ent2, rel1, rel2 = _encode(
        entity.reshape(-1), adj_entity.T.reshape(-1), adj_relation.T.reshape(-1))
    return (
        entity,
        ent1.T,
        ent2.T,
        rel1.T,
        rel2.T,
    )


# speedup vs baseline: 3.4580x; 1.1270x over previous
import functools

import jax
import jax.numpy as jnp
from jax import lax
from jax.experimental import pallas as pl
from jax.experimental.pallas import tpu as pltpu
from jax.experimental.pallas import tpu_sc as plsc

_N = 100000
_K = 8
_B = 4096
_L = 16
_NC = 2
_NS = 16
_NW = _NC * _NS
_BPW = _B // _NW
_C2 = _K * _K

_MESH = plsc.VectorSubcoreMesh(core_axis_name="c", subcore_axis_name="s")
_PARAMS = pltpu.CompilerParams(use_tc_tiling_on_sc=True)


@functools.partial(
    pl.kernel,
    mesh=_MESH,
    compiler_params=_PARAMS,
    out_type=[
        jax.ShapeDtypeStruct((_K, _B), jnp.int32),
        jax.ShapeDtypeStruct((_C2, _B), jnp.int32),
        jax.ShapeDtypeStruct((_K, _B), jnp.int32),
        jax.ShapeDtypeStruct((_C2, _B), jnp.int32),
    ],
    scratch_types=[
        pltpu.VMEM((_BPW,), jnp.int32),
        pltpu.VMEM((_K, _BPW), jnp.int32),
        pltpu.VMEM((_K, _BPW), jnp.int32),
        pltpu.VMEM((_K, _BPW), jnp.int32),
        pltpu.VMEM((_C2, _BPW), jnp.int32),
        pltpu.VMEM((_C2, _BPW), jnp.int32),
        pltpu.VMEM((_C2, _BPW), jnp.int32),
        pltpu.SemaphoreType.DMA,
        pltpu.SemaphoreType.DMA,
        pltpu.SemaphoreType.DMA,
    ],
)
def _encode(ent_hbm, adj_ef_hbm, adj_rf_hbm,
            ent1_hbm, ent2_hbm, rel1_hbm, rel2_hbm,
            idx_v, eo1_v, e1_v, r1_v, eo2_v, e2_v, r2_v,
            sem_a, sem_b, sem_w):
    wid = lax.axis_index("s") * _NC + lax.axis_index("c")
    base = pl.multiple_of(wid * _BPW, 8)

    pltpu.sync_copy(ent_hbm.at[pl.ds(base, _BPW)], idx_v)

    for t in range(_BPW // _L):
        e16 = idx_v[pl.ds(t * _L, _L)]
        for j in range(_K):
            eo1_v[j, pl.ds(t * _L, _L)] = e16 + (_N * j)

    h1 = []
    for j in range(_K):
        h1.append(pltpu.async_copy(
            adj_ef_hbm.at[eo1_v.at[j]], e1_v.at[j], sem_a))
        h1.append(pltpu.async_copy(
            adj_rf_hbm.at[eo1_v.at[j]], r1_v.at[j], sem_b))
    for cp in h1:
        cp.wait()

    w_e1 = pltpu.async_copy(e1_v, ent1_hbm.at[:, pl.ds(base, _BPW)], sem_w)
    w_r1 = pltpu.async_copy(r1_v, rel1_hbm.at[:, pl.ds(base, _BPW)], sem_w)

    def expand2(jt, carry):
        j = jt // (_BPW // _L)
        t = jt % (_BPW // _L)
        e16 = e1_v[j, pl.ds(pl.multiple_of(t * _L, _L), _L)]
        for k in range(_K):
            eo2_v[j * _K + k, pl.ds(pl.multiple_of(t * _L, _L), _L)] = (
                e16 + (_N * k))
        return carry
    lax.fori_loop(0, _K * (_BPW // _L), expand2, 0)

    def fire(c, carry):
        ic = eo2_v.at[c]
        pltpu.async_copy(adj_ef_hbm.at[ic], e2_v.at[c], sem_a)
        pltpu.async_copy(adj_rf_hbm.at[ic], r2_v.at[c], sem_b)
        return carry
    lax.fori_loop(0, _C2, fire, 0)

    def drain(c, carry):
        pltpu.make_async_copy(
            adj_ef_hbm.at[pl.ds(0, _BPW)], e2_v.at[c], sem_a).wait()
        pltpu.make_async_copy(
            adj_rf_hbm.at[pl.ds(0, _BPW)], r2_v.at[c], sem_b).wait()
        return carry
    lax.fori_loop(0, _C2, drain, 0)

    w_e2 = pltpu.async_copy(e2_v, ent2_hbm.at[:, pl.ds(base, _BPW)], sem_w)
    w_r2 = pltpu.async_copy(r2_v, rel2_hbm.at[:, pl.ds(base, _BPW)], sem_w)

    w_e1.wait()
    w_r1.wait()
    w_e2.wait()
    w_r2.wait()


def kernel(entity, adj_entity, adj_relation):
    ent1, ent2, rel1, rel2 = _encode(
        entity.reshape(-1), adj_entity.T.reshape(-1), adj_relation.T.reshape(-1))
    return (
        entity,
        ent1.T,
        ent2.T,
        rel1.T,
        rel2.T,
    )


# speedup vs baseline: 3.4629x; 1.0014x over previous
import functools

import jax
import jax.numpy as jnp
from jax import lax
from jax.experimental import pallas as pl
from jax.experimental.pallas import tpu as pltpu
from jax.experimental.pallas import tpu_sc as plsc

_N = 100000
_K = 8
_B = 4096
_L = 16
_NC = 2
_NS = 16
_NW = _NC * _NS
_BPW = _B // _NW
_C2 = _K * _K

_MESH = plsc.VectorSubcoreMesh(core_axis_name="c", subcore_axis_name="s")
_PARAMS = pltpu.CompilerParams(use_tc_tiling_on_sc=True)


@functools.partial(
    pl.kernel,
    mesh=_MESH,
    compiler_params=_PARAMS,
    out_type=[
        jax.ShapeDtypeStruct((_K, _B), jnp.int32),
        jax.ShapeDtypeStruct((_C2, _B), jnp.int32),
        jax.ShapeDtypeStruct((_K, _B), jnp.int32),
        jax.ShapeDtypeStruct((_C2, _B), jnp.int32),
    ],
    scratch_types=[
        pltpu.VMEM((_BPW,), jnp.int32),
        pltpu.VMEM((_K, _BPW), jnp.int32),
        pltpu.VMEM((_K, _BPW), jnp.int32),
        pltpu.VMEM((_K, _BPW), jnp.int32),
        pltpu.VMEM((_C2, _BPW), jnp.int32),
        pltpu.VMEM((_C2, _BPW), jnp.int32),
        pltpu.VMEM((_C2, _BPW), jnp.int32),
        pltpu.SemaphoreType.DMA,
        pltpu.SemaphoreType.DMA,
        pltpu.SemaphoreType.DMA,
    ],
)
def _encode(ent_hbm, adj_ef_hbm, adj_rf_hbm,
            ent1_hbm, ent2_hbm, rel1_hbm, rel2_hbm,
            idx_v, eo1_v, e1_v, r1_v, eo2_v, e2_v, r2_v,
            sem_a, sem_b, sem_w):
    wid = lax.axis_index("s") * _NC + lax.axis_index("c")
    base = pl.multiple_of(wid * _BPW, 8)

    pltpu.sync_copy(ent_hbm.at[pl.ds(base, _BPW)], idx_v)

    for t in range(_BPW // _L):
        e16 = idx_v[pl.ds(t * _L, _L)]
        for j in range(_K):
            eo1_v[j, pl.ds(t * _L, _L)] = e16 + (_N * j)

    h1 = []
    for j in range(_K):
        h1.append(pltpu.async_copy(
            adj_ef_hbm.at[eo1_v.at[j]], e1_v.at[j], sem_a))
        h1.append(pltpu.async_copy(
            adj_rf_hbm.at[eo1_v.at[j]], r1_v.at[j], sem_b))
    for cp in h1:
        cp.wait()

    w_e1 = pltpu.async_copy(e1_v, ent1_hbm.at[:, pl.ds(base, _BPW)], sem_w)
    w_r1 = pltpu.async_copy(r1_v, rel1_hbm.at[:, pl.ds(base, _BPW)], sem_w)

    def expand2(jt, carry):
        j = jt // (_BPW // _L)
        t = jt % (_BPW // _L)
        e16 = e1_v[j, pl.ds(pl.multiple_of(t * _L, _L), _L)]
        for k in range(_K):
            eo2_v[j * _K + k, pl.ds(pl.multiple_of(t * _L, _L), _L)] = (
                e16 + (_N * k))
        return carry
    lax.fori_loop(0, _K * (_BPW // _L), expand2, 0)

    def fire(c, carry):
        ic = eo2_v.at[c]
        pltpu.async_copy(adj_ef_hbm.at[ic], e2_v.at[c], sem_a)
        pltpu.async_copy(adj_rf_hbm.at[ic], r2_v.at[c], sem_b)
        return carry
    lax.fori_loop(0, _C2, fire, 0)

    pltpu.make_async_copy(ent2_hbm.at[:, pl.ds(0, _BPW)], e2_v, sem_a).wait()
    pltpu.make_async_copy(rel2_hbm.at[:, pl.ds(0, _BPW)], r2_v, sem_b).wait()

    w_e2 = pltpu.async_copy(e2_v, ent2_hbm.at[:, pl.ds(base, _BPW)], sem_w)
    w_r2 = pltpu.async_copy(r2_v, rel2_hbm.at[:, pl.ds(base, _BPW)], sem_w)

    w_e1.wait()
    w_r1.wait()
    w_e2.wait()
    w_r2.wait()


def kernel(entity, adj_entity, adj_relation):
    ent1, ent2, rel1, rel2 = _encode(
        entity.reshape(-1), adj_entity.T.reshape(-1), adj_relation.T.reshape(-1))
    return (
        entity,
        ent1.T,
        ent2.T,
        rel1.T,
        rel2.T,
    )
